# Initial kernel scaffold; baseline (speedup 1.0000x reference)
import jax
import jax.numpy as jnp
from jax import lax
from jax.experimental import pallas as pl
from jax.experimental.pallas import tpu as pltpu
from jax.experimental.pallas import tpu_sc as plsc

NC = 2
NS = 16
L = 16
NW = NC * NS

E = 320000
D = 128
EPW = E // NW
C = 80
NCHUNK = EPW // C


def _decoder_body(z_hbm, src_hbm, dst_hbm, out_hbm,
                  sidx, didx, srows, drows, outv, sem0, sem1):
    wid = lax.axis_index("s") * NC + lax.axis_index("c")
    lanes = lax.iota(jnp.int32, L)

    @pl.loop(0, NCHUNK)
    def _chunk(c):
        base = wid * EPW + c * C
        pltpu.sync_copy(src_hbm.at[pl.ds(base, C)], sidx)
        pltpu.sync_copy(dst_hbm.at[pl.ds(base, C)], didx)
        cp0 = pltpu.async_copy(z_hbm.at[sidx], srows, sem0)
        cp1 = pltpu.async_copy(z_hbm.at[didx], drows, sem1)
        cp0.wait()
        cp1.wait()
        for b in range(C // L):
            rows = lanes + b * L

            def _k(k, acc):
                cols = jnp.full((L,), k, jnp.int32)
                vs = plsc.load_gather(srows, [rows, cols])
                vd = plsc.load_gather(drows, [rows, cols])
                return acc + vs * vd

            acc = lax.fori_loop(0, D, _k, jnp.zeros((L,), jnp.float32))
            outv[pl.ds(b * L, L)] = 1.0 / (1.0 + jnp.exp(-acc))
        pltpu.sync_copy(outv, out_hbm.at[pl.ds(base, C)])


@jax.jit
def _run(z, src, dst):
    mesh = plsc.VectorSubcoreMesh(
        core_axis_name="c", subcore_axis_name="s",
        num_cores=NC, num_subcores=NS)
    f = pl.kernel(
        _decoder_body,
        out_type=jax.ShapeDtypeStruct((E,), jnp.float32),
        mesh=mesh,
        scratch_types=[
            pltpu.VMEM((C,), jnp.int32),
            pltpu.VMEM((C,), jnp.int32),
            pltpu.VMEM((C, D), jnp.float32),
            pltpu.VMEM((C, D), jnp.float32),
            pltpu.VMEM((C,), jnp.float32),
            pltpu.SemaphoreType.DMA,
            pltpu.SemaphoreType.DMA,
        ],
    )
    return f(z, src, dst)


def kernel(z, edge_index):
    ei = edge_index.astype(jnp.int32)
    return _run(z, ei[0], ei[1])


# speedup vs baseline: 1.0990x; 1.0990x over previous
import jax
import jax.numpy as jnp
from jax import lax
from jax.experimental import pallas as pl
from jax.experimental.pallas import tpu as pltpu
from jax.experimental.pallas import tpu_sc as plsc

NC = 2
NS = 16
L = 16
NW = NC * NS

E = 320000
D = 128
EPW = E // NW
C = 80
NCHUNK = EPW // C


def _decoder_body(z_hbm, src_hbm, dst_hbm, out_hbm,
                  sidx, didx, srows, drows, outv, sem0, sem1):
    wid = lax.axis_index("s") * NC + lax.axis_index("c")
    lanes = lax.iota(jnp.int32, L)

    @pl.loop(0, NCHUNK)
    def _chunk(c):
        base = wid * EPW + c * C
        pltpu.sync_copy(src_hbm.at[pl.ds(base, C)], sidx)
        pltpu.sync_copy(dst_hbm.at[pl.ds(base, C)], didx)
        cp0 = pltpu.async_copy(z_hbm.at[sidx], srows, sem0)
        cp1 = pltpu.async_copy(z_hbm.at[didx], drows, sem1)
        cp0.wait()
        cp1.wait()
        for b in range(C // L):
            rows = lanes + b * L

            def _k(k, acc):
                cols = jnp.full((L,), k, jnp.int32)
                vs = plsc.load_gather(srows, [rows, cols])
                vd = plsc.load_gather(drows, [rows, cols])
                return acc + vs * vd

            acc = lax.fori_loop(0, D, _k, jnp.zeros((L,), jnp.float32))
            outv[pl.ds(b * L, L)] = 1.0 / (1.0 + jnp.exp(-acc))
        pltpu.sync_copy(outv, out_hbm.at[pl.ds(base, C)])


@jax.jit
def _run(z, src, dst):
    mesh = plsc.VectorSubcoreMesh(
        core_axis_name="c", subcore_axis_name="s",
        num_cores=NC, num_subcores=NS)
    f = pl.kernel(
        _decoder_body,
        out_type=jax.ShapeDtypeStruct((E,), jnp.float32),
        mesh=mesh,
        scratch_types=[
            pltpu.VMEM((C,), jnp.int32),
            pltpu.VMEM((C,), jnp.int32),
            pltpu.VMEM((C, D), jnp.float32),
            pltpu.VMEM((C, D), jnp.float32),
            pltpu.VMEM((C,), jnp.float32),
            pltpu.SemaphoreType.DMA,
            pltpu.SemaphoreType.DMA,
        ],
        compiler_params=pltpu.CompilerParams(needs_layout_passes=False),
    )
    return f(z, src, dst)


def kernel(z, edge_index):
    ei = edge_index.astype(jnp.int32)
    return _run(z, ei[0], ei[1])
